# baseline (speedup 1.0000x reference)
import numpy as np
import jax
import jax.numpy as jnp
from jax import lax
from jax.experimental import pallas as pl
from jax.experimental.pallas import tpu as pltpu
from jax.experimental.pallas import tpu_sc as plsc

_D = 2048
_H = 512
_B = 4
_N = 4096
_TN = 1024
_G = (_B * _N) // _TN
_K_RANK = 3277

_INTERPRET = False

_SQRT_HALF = np.float32(np.sqrt(0.5))

_ERFC_P = [2.326819970068386e-2, -1.387039388740657e-1, 3.687424674597105e-1,
           -5.824733027278666e-1, 6.210004764949774e-1, -4.944515323274145e-1,
           3.404879937665872e-1, -2.741127028184656e-1, 5.638259427386472e-1]
_ERFC_R = [-1.047766399936249e+1, 1.297719955372516e+1, -7.495518717768503e+0,
           2.921019019210786e+0, -1.015265279202700e+0, 4.218463358204948e-1,
           -2.820767439740514e-1, 5.641895067754075e-1]
_ERF_T = [7.853861353153693e-5, -8.010193625184903e-4, 5.188327685732524e-3,
          -2.685381193529856e-2, 1.128358514861418e-1, -3.761262582423300e-1,
          1.128379165726710e+0]
_MAXLOG = np.float32(88.72283905206835)


def _poly(y, coefs):
    p = jnp.zeros_like(y)
    for c in coefs:
        p = p * y + np.float32(c)
    return p


def _erfc_f32(x):
    abs_x = jnp.abs(x)
    w = -(x * x)
    z = jnp.exp(w)
    q = 1.0 / abs_x
    y2 = q * q
    small = abs_x < 2.0
    p = jnp.zeros_like(x)
    for cp, cr in zip(_ERFC_P, [0.0] + _ERFC_R):
        p = p * y2 + jnp.where(small, np.float32(cp), np.float32(cr))
    yv = z * q * p
    y_clamp = jnp.where(w < -_MAXLOG, 0.0, yv)
    erfc_big = jnp.where(x < 0.0, 2.0 - y_clamp, y_clamp)
    erf_small = x * _poly(x * x, _ERF_T)
    return jnp.where(abs_x > 1.0, erfc_big, 1.0 - erf_small)


def _score_body(x_ref, w1t_ref, b1_ref, w2p_ref, b2_ref, t_ref, o_ref):
    x = x_ref[0]
    h = jnp.dot(x.astype(jnp.bfloat16), w1t_ref[...].astype(jnp.bfloat16),
                preferred_element_type=jnp.float32)
    h = h + b1_ref[...]
    g = 0.5 * h * _erfc_f32(h * -_SQRT_HALF)
    logit = jnp.dot(g.astype(jnp.bfloat16), w2p_ref[...].astype(jnp.bfloat16),
                    preferred_element_type=jnp.float32)[:, 0]
    logit = logit + b2_ref[0, 0]
    o_ref[0, 0] = (1.0 / (1.0 + jnp.exp(-logit))) * t_ref[0, 0, 0]


def _sc_thr_body(keys_hbm, rare_hbm, mask_hbm, keys_v, rare_v, hist_v):
    c = lax.axis_index("c")
    s_id = lax.axis_index("s")
    b = c * 2 + s_id // 8
    is_master = (s_id % 8) == 0

    @pl.when(is_master)
    def _():
        pltpu.sync_copy(keys_hbm.at[b], keys_v)
        pltpu.sync_copy(rare_hbm.at[b], rare_v)

        lanes = lax.iota(jnp.int32, 16)
        ones = jnp.ones((16,), jnp.int32)
        zeros = jnp.zeros((16,), jnp.int32)

        def _scalarize(v):
            return jnp.max(v) if getattr(v, "ndim", 0) else v

        krem = jnp.int32(_K_RANK)
        prefix = jnp.int32(0)
        for shift, bits in ((22, 8), (14, 8), (6, 8), (0, 6)):
            nbins = 1 << bits

            def zero_body(i, carry):
                hist_v[pl.ds(i * 16, 16)] = zeros
                return carry
            lax.fori_loop(0, nbins, zero_body, 0)

            pshift = shift + bits

            def build_body(i, carry):
                sl = pl.ds(i * 16, 16)
                k = keys_v[sl]
                m = (k >> pshift) == prefix
                d = (k >> shift) & (nbins - 1)
                plsc.addupdate_scatter(hist_v, [d * 16 + lanes], ones, mask=m)
                return carry
            lax.fori_loop(0, _N // 16, build_body, 0)

            def chunk_body(j, ctot):
                acc = zeros
                for u in range(16):
                    acc = acc + hist_v[pl.ds(j * 256 + u * 16, 16)]
                return jnp.where(lanes == j, jnp.sum(acc), ctot)
            ctot = lax.fori_loop(0, nbins // 16, chunk_body, zeros)
            ccum = plsc.cumsum(ctot)
            cidx = _scalarize(plsc.all_reduce_ffs(ccum >= krem))
            krem = krem - jnp.sum(jnp.where(lanes == cidx, ccum - ctot, zeros))

            def bin_body(u, btot):
                v = hist_v[pl.ds((cidx * 16 + u) * 16, 16)]
                return jnp.where(lanes == u, jnp.sum(v), btot)
            btot = lax.fori_loop(0, 16, bin_body, zeros)
            bcum = plsc.cumsum(btot)
            bidx = _scalarize(plsc.all_reduce_ffs(bcum >= krem))
            krem = krem - jnp.sum(jnp.where(lanes == bidx, bcum - btot, zeros))
            prefix = (prefix << bits) | (cidx * 16 + bidx)

        thr_key = prefix

        def mask_body(i, carry):
            sl = pl.ds(i * 16, 16)
            keep = (keys_v[sl] > thr_key) & (rare_v[sl] == 0)
            rare_v[sl] = keep.astype(jnp.int32)
            return carry
        lax.fori_loop(0, _N // 16, mask_body, 0)
        pltpu.sync_copy(rare_v, mask_hbm.at[b])


def kernel(token_repr, t_frac, rare_mask, W1, b1, W2, b2):
    x = token_repr.reshape(_G, _TN, _D)
    w1t = W1.T
    b1r = b1.reshape(1, _H)
    w2p = jnp.zeros((_H, 128), W2.dtype).at[:, 0].set(W2[0])
    b2r = b2.reshape(1, 1)

    t_tile = jnp.repeat(t_frac, _N // _TN).reshape(_G, 1, 1)

    scores = pl.pallas_call(
        _score_body,
        grid=(_G,),
        in_specs=[
            pl.BlockSpec((1, _TN, _D), lambda i: (i, 0, 0)),
            pl.BlockSpec((_D, _H), lambda i: (0, 0)),
            pl.BlockSpec((1, _H), lambda i: (0, 0)),
            pl.BlockSpec((_H, 128), lambda i: (0, 0)),
            pl.BlockSpec(memory_space=pltpu.SMEM),
            pl.BlockSpec((1, 1, 1), lambda i: (i, 0, 0), memory_space=pltpu.SMEM),
        ],
        out_specs=pl.BlockSpec((1, 1, _TN), lambda i: (i, 0, 0)),
        out_shape=jax.ShapeDtypeStruct((_G, 1, _TN), jnp.float32),
        interpret=_INTERPRET,
    )(x, w1t, b1r, w2p, b2r, t_tile)

    scores = scores.reshape(_B, _N)
    keys = lax.bitcast_convert_type(scores, jnp.int32)
    rare_i = rare_mask.astype(jnp.int32)

    mask_i = pl.kernel(
        _sc_thr_body,
        out_type=jax.ShapeDtypeStruct((_B, _N), jnp.int32),
        mesh=plsc.VectorSubcoreMesh(core_axis_name="c", subcore_axis_name="s"),
        compiler_params=pltpu.CompilerParams(needs_layout_passes=False),
        scratch_types=[
            pltpu.VMEM((_N,), jnp.int32),
            pltpu.VMEM((_N,), jnp.int32),
            pltpu.VMEM((_N,), jnp.int32),
        ],
    )(keys, rare_i)

    return mask_i.astype(jnp.bool_), scores


# speedup vs baseline: 1.0357x; 1.0357x over previous
import numpy as np
import jax
import jax.numpy as jnp
from jax import lax
from jax.experimental import pallas as pl
from jax.experimental.pallas import tpu as pltpu
from jax.experimental.pallas import tpu_sc as plsc

_D = 2048
_H = 512
_B = 4
_N = 4096
_TN = 1024
_G = (_B * _N) // _TN
_K_RANK = 3277

_INTERPRET = False

_SQRT_HALF = np.float32(np.sqrt(0.5))

_ERFC_P = [2.326819970068386e-2, -1.387039388740657e-1, 3.687424674597105e-1,
           -5.824733027278666e-1, 6.210004764949774e-1, -4.944515323274145e-1,
           3.404879937665872e-1, -2.741127028184656e-1, 5.638259427386472e-1]
_ERFC_R = [-1.047766399936249e+1, 1.297719955372516e+1, -7.495518717768503e+0,
           2.921019019210786e+0, -1.015265279202700e+0, 4.218463358204948e-1,
           -2.820767439740514e-1, 5.641895067754075e-1]
_ERF_T = [7.853861353153693e-5, -8.010193625184903e-4, 5.188327685732524e-3,
          -2.685381193529856e-2, 1.128358514861418e-1, -3.761262582423300e-1,
          1.128379165726710e+0]
_MAXLOG = np.float32(88.72283905206835)


def _poly(y, coefs):
    p = jnp.zeros_like(y)
    for c in coefs:
        p = p * y + np.float32(c)
    return p


def _erfc_f32(x):
    abs_x = jnp.abs(x)
    w = -(x * x)
    z = jnp.exp(w)
    q = 1.0 / abs_x
    y2 = q * q
    small = abs_x < 2.0
    p = jnp.zeros_like(x)
    for cp, cr in zip(_ERFC_P, [0.0] + _ERFC_R):
        p = p * y2 + jnp.where(small, np.float32(cp), np.float32(cr))
    yv = z * q * p
    y_clamp = jnp.where(w < -_MAXLOG, 0.0, yv)
    erfc_big = jnp.where(x < 0.0, 2.0 - y_clamp, y_clamp)
    erf_small = x * _poly(x * x, _ERF_T)
    return jnp.where(abs_x > 1.0, erfc_big, 1.0 - erf_small)


def _score_body(x_ref, w1t_ref, b1_ref, w2p_ref, b2_ref, t_ref, o_ref):
    x = x_ref[0]
    h = jnp.dot(x.astype(jnp.bfloat16), w1t_ref[...].astype(jnp.bfloat16),
                preferred_element_type=jnp.float32)
    h = h + b1_ref[...]
    g = 0.5 * h * _erfc_f32(h * -_SQRT_HALF)
    logit = jnp.dot(g.astype(jnp.bfloat16), w2p_ref[...].astype(jnp.bfloat16),
                    preferred_element_type=jnp.float32)[:, 0]
    logit = logit + b2_ref[0, 0]
    o_ref[0, 0] = (1.0 / (1.0 + jnp.exp(-logit))) * t_ref[0, 0, 0]


def _sc_thr_body(keys_hbm, rare_hbm, mask_hbm, keys_v, rare_v, hist_v):
    c = lax.axis_index("c")
    s_id = lax.axis_index("s")
    b = c * 2 + s_id // 8
    is_master = (s_id % 8) == 0

    @pl.when(is_master)
    def _():
        pltpu.sync_copy(keys_hbm.at[b], keys_v)
        pltpu.sync_copy(rare_hbm.at[b], rare_v)

        lanes = lax.iota(jnp.int32, 16)
        ones = jnp.ones((16,), jnp.int32)
        zeros = jnp.zeros((16,), jnp.int32)

        def _scalarize(v):
            return jnp.max(v) if getattr(v, "ndim", 0) else v

        _U = 8

        krem = jnp.int32(_K_RANK)
        prefix = jnp.int32(0)
        for shift, bits in ((22, 8), (14, 8), (6, 8), (0, 6)):
            nbins = 1 << bits

            def zero_body(i, carry):
                for u in range(_U):
                    hist_v[pl.ds((i * _U + u) * 16, 16)] = zeros
                return carry
            lax.fori_loop(0, nbins // _U, zero_body, 0)

            pshift = shift + bits
            top = shift == 22

            def build_body(i, carry):
                for u in range(_U):
                    k = keys_v[pl.ds((i * _U + u) * 16, 16)]
                    m = None if top else (k >> pshift) == prefix
                    d = (k >> shift) & (nbins - 1)
                    plsc.addupdate_scatter(hist_v, [d * 16 + lanes], ones, mask=m)
                return carry
            lax.fori_loop(0, _N // 16 // _U, build_body, 0)

            def chunk_body(j, ctot):
                acc = zeros
                for u in range(16):
                    acc = acc + hist_v[pl.ds(j * 256 + u * 16, 16)]
                return jnp.where(lanes == j, jnp.sum(acc), ctot)
            ctot = lax.fori_loop(0, nbins // 16, chunk_body, zeros)
            ccum = plsc.cumsum(ctot)
            cidx = _scalarize(plsc.all_reduce_ffs(ccum >= krem))
            krem = krem - jnp.sum(jnp.where(lanes == cidx, ccum - ctot, zeros))

            def bin_body(u, btot):
                v = hist_v[pl.ds((cidx * 16 + u) * 16, 16)]
                return jnp.where(lanes == u, jnp.sum(v), btot)
            btot = lax.fori_loop(0, 16, bin_body, zeros)
            bcum = plsc.cumsum(btot)
            bidx = _scalarize(plsc.all_reduce_ffs(bcum >= krem))
            krem = krem - jnp.sum(jnp.where(lanes == bidx, bcum - btot, zeros))
            prefix = (prefix << bits) | (cidx * 16 + bidx)

        thr_key = prefix

        def mask_body(i, carry):
            for u in range(_U):
                sl = pl.ds((i * _U + u) * 16, 16)
                keep = (keys_v[sl] > thr_key) & (rare_v[sl] == 0)
                rare_v[sl] = keep.astype(jnp.int32)
            return carry
        lax.fori_loop(0, _N // 16 // _U, mask_body, 0)
        pltpu.sync_copy(rare_v, mask_hbm.at[b])


def kernel(token_repr, t_frac, rare_mask, W1, b1, W2, b2):
    x = token_repr.reshape(_G, _TN, _D)
    w1t = W1.T
    b1r = b1.reshape(1, _H)
    w2p = jnp.zeros((_H, 128), W2.dtype).at[:, 0].set(W2[0])
    b2r = b2.reshape(1, 1)

    t_tile = jnp.repeat(t_frac, _N // _TN).reshape(_G, 1, 1)

    scores = pl.pallas_call(
        _score_body,
        grid=(_G,),
        in_specs=[
            pl.BlockSpec((1, _TN, _D), lambda i: (i, 0, 0)),
            pl.BlockSpec((_D, _H), lambda i: (0, 0)),
            pl.BlockSpec((1, _H), lambda i: (0, 0)),
            pl.BlockSpec((_H, 128), lambda i: (0, 0)),
            pl.BlockSpec(memory_space=pltpu.SMEM),
            pl.BlockSpec((1, 1, 1), lambda i: (i, 0, 0), memory_space=pltpu.SMEM),
        ],
        out_specs=pl.BlockSpec((1, 1, _TN), lambda i: (i, 0, 0)),
        out_shape=jax.ShapeDtypeStruct((_G, 1, _TN), jnp.float32),
        interpret=_INTERPRET,
    )(x, w1t, b1r, w2p, b2r, t_tile)

    scores = scores.reshape(_B, _N)
    keys = lax.bitcast_convert_type(scores, jnp.int32)
    rare_i = rare_mask.astype(jnp.int32)

    mask_i = pl.kernel(
        _sc_thr_body,
        out_type=jax.ShapeDtypeStruct((_B, _N), jnp.int32),
        mesh=plsc.VectorSubcoreMesh(core_axis_name="c", subcore_axis_name="s"),
        compiler_params=pltpu.CompilerParams(needs_layout_passes=False),
        scratch_types=[
            pltpu.VMEM((_N,), jnp.int32),
            pltpu.VMEM((_N,), jnp.int32),
            pltpu.VMEM((_N,), jnp.int32),
        ],
    )(keys, rare_i)

    return mask_i.astype(jnp.bool_), scores
